# Initial kernel scaffold; baseline (speedup 1.0000x reference)
import jax
import jax.numpy as jnp
from jax.experimental import pallas as pl


def kernel(x, table, W1, b1, W2, b2):
    raise NotImplementedError("write your pallas kernel here")


# speedup vs baseline: 7.4152x; 7.4152x over previous
import functools

import jax
import jax.numpy as jnp
from jax import lax
from jax.experimental import pallas as pl
from jax.experimental.pallas import tpu as pltpu
from jax.experimental.pallas import tpu_sc as plsc

V = 100000
D = 128
H = 256
NP = 1000
NPP = 1024
B = 4096
GROUP = 200


def _make_pool_kernel():
    info = plsc.get_sparse_core_info()
    nc, ns = info.num_cores, info.num_subcores
    nw = nc * ns
    assert B % nw == 0
    bpw = B // nw
    half = GROUP // 2

    mesh = plsc.VectorSubcoreMesh(core_axis_name="c", subcore_axis_name="s")

    @functools.partial(
        pl.kernel,
        mesh=mesh,
        out_type=jax.ShapeDtypeStruct((B, D), jnp.float32),
        scratch_types=[
            pltpu.VMEM((2, half), jnp.int32),
            pltpu.VMEM((GROUP, D), jnp.float32),
            pltpu.VMEM((bpw, D), jnp.float32),
            pltpu.SemaphoreType.DMA,
        ],
    )
    def pool(idx_hbm, table_hbm, out_hbm, idx_v, rows_v, out_v, sem):
        wid = lax.axis_index("s") * nc + lax.axis_index("c")
        base = wid * bpw

        def body(e, _):
            pltpu.sync_copy(idx_hbm.at[base + e], idx_v)
            cp0 = pltpu.async_copy(
                table_hbm.at[idx_v.at[0]], rows_v.at[pl.ds(0, half)], sem)
            cp1 = pltpu.async_copy(
                table_hbm.at[idx_v.at[1]], rows_v.at[pl.ds(half, half)], sem)
            cp0.wait()
            cp1.wait()

            def rbody(r, accs):
                return tuple(
                    accs[c] + rows_v[r, pl.ds(c * 16, 16)] for c in range(8))

            accs = lax.fori_loop(
                0, GROUP, rbody,
                tuple(jnp.zeros((16,), jnp.float32) for _ in range(8)))
            for c in range(8):
                out_v[e, pl.ds(c * 16, 16)] = accs[c]
            return 0

        lax.fori_loop(0, bpw, body, 0)
        pltpu.sync_copy(out_v, out_hbm.at[pl.ds(base, bpw)])

    return pool


def _mlp_body(s_ref, w1_ref, b1_ref, w2_ref, b2_ref, out_ref):
    s = s_ref[...]
    h = jax.nn.sigmoid(
        jnp.dot(s, w1_ref[...], preferred_element_type=jnp.float32)
        + b1_ref[...])
    logits = (jnp.dot(h, w2_ref[...], preferred_element_type=jnp.float32)
              + b2_ref[...])
    m = jnp.max(logits, axis=-1, keepdims=True)
    lse = jnp.log(jnp.sum(jnp.exp(logits - m), axis=-1, keepdims=True)) + m
    out_ref[...] = logits - lse


def _mlp(pooled, w1, b1, w2p, b2p):
    bm = 512
    grid = (B // bm,)
    return pl.pallas_call(
        _mlp_body,
        grid=grid,
        in_specs=[
            pl.BlockSpec((bm, D), lambda i: (i, 0)),
            pl.BlockSpec((D, H), lambda i: (0, 0)),
            pl.BlockSpec((1, H), lambda i: (0, 0)),
            pl.BlockSpec((H, NPP), lambda i: (0, 0)),
            pl.BlockSpec((1, NPP), lambda i: (0, 0)),
        ],
        out_specs=pl.BlockSpec((bm, NPP), lambda i: (i, 0)),
        out_shape=jax.ShapeDtypeStruct((B, NPP), jnp.float32),
    )(pooled, w1, b1, w2p, b2p)


def kernel(x, table, W1, b1, W2, b2):
    idx = x.reshape(B, 2, GROUP // 2)
    pooled = _make_pool_kernel()(idx, table)
    w2p = jnp.pad(W2, ((0, 0), (0, NPP - NP)))
    b2p = jnp.pad(b2, (0, NPP - NP), constant_values=-1e30)
    out = _mlp(pooled, W1, b1.reshape(1, H), w2p, b2p.reshape(1, NPP))
    return out[:, :NP]


# speedup vs baseline: 14.4419x; 1.9476x over previous
import functools

import jax
import jax.numpy as jnp
from jax import lax
from jax.experimental import pallas as pl
from jax.experimental.pallas import tpu as pltpu
from jax.experimental.pallas import tpu_sc as plsc

V = 100000
D = 128
H = 256
NP = 1000
NPP = 1024
B = 4096
GROUP = 200


def _make_pool_kernel():
    info = plsc.get_sparse_core_info()
    nc, ns = info.num_cores, info.num_subcores
    nw = nc * ns
    assert B % nw == 0
    bpw = B // nw
    half = GROUP // 2

    mesh = plsc.VectorSubcoreMesh(core_axis_name="c", subcore_axis_name="s")

    @functools.partial(
        pl.kernel,
        mesh=mesh,
        out_type=jax.ShapeDtypeStruct((B, D), jnp.float32),
        scratch_types=[
            pltpu.VMEM((bpw, 2, half), jnp.int32),
            pltpu.VMEM((2, GROUP, D), jnp.float32),
            pltpu.VMEM((bpw, D), jnp.float32),
            pltpu.SemaphoreType.DMA,
            pltpu.SemaphoreType.DMA,
        ],
    )
    def pool(idx_hbm, table_hbm, out_hbm, idx_v, rows_v, out_v, sem0, sem1):
        wid = lax.axis_index("s") * nc + lax.axis_index("c")
        base = wid * bpw
        sems = (sem0, sem1)

        pltpu.sync_copy(idx_hbm.at[pl.ds(base, bpw)], idx_v)

        def fire(row, slot):
            pltpu.async_copy(table_hbm.at[idx_v.at[row, 0]],
                             rows_v.at[slot, pl.ds(0, half)], sems[slot])
            pltpu.async_copy(table_hbm.at[idx_v.at[row, 1]],
                             rows_v.at[slot, pl.ds(half, half)], sems[slot])

        def drain(row, slot):
            pltpu.make_async_copy(table_hbm.at[idx_v.at[row, 0]],
                                  rows_v.at[slot, pl.ds(0, half)],
                                  sems[slot]).wait()
            pltpu.make_async_copy(table_hbm.at[idx_v.at[row, 1]],
                                  rows_v.at[slot, pl.ds(half, half)],
                                  sems[slot]).wait()

        fire(0, 0)
        fire(1, 1)

        def body(i, _):
            e = i * 2
            for slot in range(2):
                row = e + slot
                drain(row, slot)

                def rbody(r4, accs):
                    r = r4 * 4
                    return tuple(
                        accs[c]
                        + (rows_v[slot, r, pl.ds(c * 16, 16)]
                           + rows_v[slot, r + 1, pl.ds(c * 16, 16)])
                        + (rows_v[slot, r + 2, pl.ds(c * 16, 16)]
                           + rows_v[slot, r + 3, pl.ds(c * 16, 16)])
                        for c in range(8))

                accs = lax.fori_loop(
                    0, GROUP // 4, rbody,
                    tuple(jnp.zeros((16,), jnp.float32) for _ in range(8)))
                for c in range(8):
                    out_v[row, pl.ds(c * 16, 16)] = accs[c]

                @pl.when(row + 2 < bpw)
                def _():
                    fire(row + 2, slot)
            return 0

        lax.fori_loop(0, bpw // 2, body, 0)
        pltpu.sync_copy(out_v, out_hbm.at[pl.ds(base, bpw)])

    return pool


def _mlp_body(s_ref, w1_ref, b1_ref, w2_ref, b2_ref, out_ref):
    s = s_ref[...]
    h = jax.nn.sigmoid(
        jnp.dot(s, w1_ref[...], preferred_element_type=jnp.float32)
        + b1_ref[...])
    logits = (jnp.dot(h, w2_ref[...], preferred_element_type=jnp.float32)
              + b2_ref[...])
    m = jnp.max(logits, axis=-1, keepdims=True)
    lse = jnp.log(jnp.sum(jnp.exp(logits - m), axis=-1, keepdims=True)) + m
    out_ref[...] = logits - lse


def _mlp(pooled, w1, b1, w2p, b2p):
    bm = 512
    grid = (B // bm,)
    return pl.pallas_call(
        _mlp_body,
        grid=grid,
        in_specs=[
            pl.BlockSpec((bm, D), lambda i: (i, 0)),
            pl.BlockSpec((D, H), lambda i: (0, 0)),
            pl.BlockSpec((1, H), lambda i: (0, 0)),
            pl.BlockSpec((H, NPP), lambda i: (0, 0)),
            pl.BlockSpec((1, NPP), lambda i: (0, 0)),
        ],
        out_specs=pl.BlockSpec((bm, NPP), lambda i: (i, 0)),
        out_shape=jax.ShapeDtypeStruct((B, NPP), jnp.float32),
    )(pooled, w1, b1, w2p, b2p)


def kernel(x, table, W1, b1, W2, b2):
    idx = x.reshape(B, 2, GROUP // 2)
    pooled = _make_pool_kernel()(idx, table)
    w2p = jnp.pad(W2, ((0, 0), (0, NPP - NP)))
    b2p = jnp.pad(b2, (0, NPP - NP), constant_values=-1e30)
    out = _mlp(pooled, W1, b1.reshape(1, H), w2p, b2p.reshape(1, NPP))
    return out[:, :NP]
